# Initial kernel scaffold; baseline (speedup 1.0000x reference)
import jax
import jax.numpy as jnp
from jax.experimental import pallas as pl


def kernel(inp, params):
    raise NotImplementedError("write your pallas kernel here")


# speedup vs baseline: 2.7444x; 2.7444x over previous
import numpy as np
import jax
import jax.numpy as jnp
from jax.experimental import pallas as pl
from jax.experimental.pallas import tpu as pltpu

B = 256
H = 96
NP = 88
EP = 1624
NUM_STEPS = 4


def _build_edges_np():
    idx = np.arange(81).reshape(9, 9)
    es = set()
    for i in range(9):
        for a in idx[i, :]:
            for b in idx[i, :]:
                if a != b:
                    es.add((int(a), int(b)))
        for a in idx[:, i]:
            for b in idx[:, i]:
                if a != b:
                    es.add((int(a), int(b)))
    for i in range(3):
        for j in range(3):
            v = idx[3 * i:3 * i + 3, 3 * j:3 * j + 3].reshape(-1)
            for a in v:
                for b in v:
                    if a != b:
                        es.add((int(a), int(b)))
    return np.array(sorted(es), dtype=np.int32)


_EDGES = _build_edges_np()
_NE = _EDGES.shape[0]

_GSRC = np.zeros((EP, NP), np.float32)
_GDST = np.zeros((EP, NP), np.float32)
_GSRC[np.arange(_NE), _EDGES[:, 0]] = 1.0
_GDST[np.arange(_NE), _EDGES[:, 1]] = 1.0
_SCAT = _GDST.T.copy()

_ROWS = np.repeat(np.arange(9), 9)
_COLS = np.tile(np.arange(9), 9)


def _rrn_kernel(oh_ref, gsrc_ref, gdst_ref, scat_ref,
                vt_ref, pc_ref, wx2_ref, bx2_ref, wx3_ref, bx3_ref,
                wx4_ref, bx4_ref,
                w1a_ref, w1b_ref, b1_ref, w2_ref, b2_ref, w3_ref, b3_ref,
                w4_ref, b4_ref,
                wl1a_ref, wl1b_ref, bl1_ref, wl2_ref, bl2_ref, wl3_ref,
                bl3_ref, wl4_ref, bl4_ref,
                wih0_ref, wih1_ref, wih2_ref, wih3_ref,
                whh0_ref, whh1_ref, whh2_ref, whh3_ref,
                bg0_ref, bg1_ref, bg2_ref, bg3_ref,
                ro_ref, rb_ref, out_ref):
    oh = oh_ref[0]

    x = oh @ vt_ref[...] + pc_ref[...]
    x = jnp.maximum(x, 0.0)
    x = jnp.maximum(x @ wx2_ref[...] + bx2_ref[...], 0.0)
    x = jnp.maximum(x @ wx3_ref[...] + bx3_ref[...], 0.0)
    x = x @ wx4_ref[...] + bx4_ref[...]

    gsrc = gsrc_ref[...]
    gdst = gdst_ref[...]
    scat = scat_ref[...]

    h = x
    c = jnp.zeros_like(x)
    wih = (wih0_ref[...], wih1_ref[...], wih2_ref[...], wih3_ref[...])
    whh = (whh0_ref[...], whh1_ref[...], whh2_ref[...], whh3_ref[...])
    bg = (bg0_ref[...], bg1_ref[...], bg2_ref[...], bg3_ref[...])

    for t in range(NUM_STEPS):
        a = h @ w1a_ref[...]
        bv = h @ w1b_ref[...]
        e = gsrc @ a + gdst @ bv + b1_ref[...]
        e = jnp.maximum(e, 0.0)
        e = jnp.maximum(e @ w2_ref[...] + b2_ref[...], 0.0)
        e = jnp.maximum(e @ w3_ref[...] + b3_ref[...], 0.0)
        e = e @ w4_ref[...] + b4_ref[...]
        fm = scat @ e

        l = fm @ wl1a_ref[...] + x @ wl1b_ref[...] + bl1_ref[...]
        l = jnp.maximum(l, 0.0)
        l = jnp.maximum(l @ wl2_ref[...] + bl2_ref[...], 0.0)
        l = jnp.maximum(l @ wl3_ref[...] + bl3_ref[...], 0.0)
        l = l @ wl4_ref[...] + bl4_ref[...]

        def gate(k):
            g = l @ wih[k] + bg[k]
            if t > 0:
                g = g + h @ whh[k]
            return g

        i_g = jax.nn.sigmoid(gate(0))
        f_g = jax.nn.sigmoid(gate(1))
        g_g = jnp.tanh(gate(2))
        o_g = jax.nn.sigmoid(gate(3))
        c = f_g * c + i_g * g_g
        h = o_g * jnp.tanh(c)

        out_ref[t, 0] = h @ ro_ref[...] + rb_ref[...]


def kernel(inp, params):
    f32 = jnp.float32

    e1w, e1b = params['e1']
    e2w, e2b = params['e2']
    e3w, e3b = params['e3']
    wx1, bx1 = params['ex'][0]
    vt = e1w @ wx1[:16]
    vt = jnp.pad(vt, ((0, 6), (0, 0)))
    emb2 = e2w[_ROWS] + e2b
    emb3 = e3w[_COLS] + e3b
    pc = emb2 @ wx1[16:32] + emb3 @ wx1[32:48] + bx1 + e1b @ wx1[:16]
    pc = jnp.pad(pc, ((0, NP - 81), (0, 0)))

    (wx2, bx2), (wx3, bx3), (wx4, bx4) = params['ex'][1:]

    (w1, b1), (w2, b2), (w3, b3), (w4, b4) = params['mm']
    w1a, w1b = w1[:H], w1[H:]
    (wl1, bl1), (wl2, bl2), (wl3, bl3), (wl4, bl4) = params['ml']
    wl1a, wl1b = wl1[:H], wl1[H:]

    wih = params['wih']
    whh = params['whh']
    bgsum = params['bih'] + params['bhh']
    wih_k = [wih[:, k * H:(k + 1) * H] for k in range(4)]
    whh_k = [whh[:, k * H:(k + 1) * H] for k in range(4)]
    bg_k = [bgsum[k * H:(k + 1) * H].reshape(1, H) for k in range(4)]

    row = lambda v: v.reshape(1, -1)

    ro = jnp.pad(params['ro'][0], ((0, 0), (0, 6)))
    rb = jnp.pad(params['ro'][1], ((0, 6),)).reshape(1, 16)

    oh = jax.nn.one_hot(inp, 10, dtype=f32)
    oh = jnp.pad(oh, ((0, 0), (0, NP - 81), (0, 6)))

    gsrc = jnp.asarray(_GSRC)
    gdst = jnp.asarray(_GDST)
    scat = jnp.asarray(_SCAT)

    def cspec(shape):
        nd = len(shape)
        return pl.BlockSpec(shape, lambda i, _n=nd: (0,) * _n)

    operands = [
        oh, gsrc, gdst, scat,
        vt, pc, wx2, row(bx2), wx3, row(bx3), wx4, row(bx4),
        w1a, w1b, row(b1), w2, row(b2), w3, row(b3), w4, row(b4),
        wl1a, wl1b, row(bl1), wl2, row(bl2), wl3, row(bl3), wl4, row(bl4),
        *wih_k, *whh_k, *bg_k,
        ro, rb,
    ]
    in_specs = [pl.BlockSpec((1, NP, 16), lambda i: (i, 0, 0))]
    in_specs += [cspec(op.shape) for op in operands[1:]]

    out = pl.pallas_call(
        _rrn_kernel,
        grid=(B,),
        in_specs=in_specs,
        out_specs=pl.BlockSpec((NUM_STEPS, 1, NP, 16), lambda i: (0, i, 0, 0)),
        out_shape=jax.ShapeDtypeStruct((NUM_STEPS, B, NP, 16), f32),
        compiler_params=pltpu.CompilerParams(
            dimension_semantics=("arbitrary",),
        ),
    )(*operands)

    return out[:, :, :81, :10].reshape(NUM_STEPS, B * 81, 10)


# speedup vs baseline: 2.7466x; 1.0008x over previous
import numpy as np
import jax
import jax.numpy as jnp
from jax.experimental import pallas as pl
from jax.experimental.pallas import tpu as pltpu

B = 256
H = 96
NP = 88
EP = 1624
NUM_STEPS = 4


def _build_edges_np():
    idx = np.arange(81).reshape(9, 9)
    es = set()
    for i in range(9):
        for a in idx[i, :]:
            for b in idx[i, :]:
                if a != b:
                    es.add((int(a), int(b)))
        for a in idx[:, i]:
            for b in idx[:, i]:
                if a != b:
                    es.add((int(a), int(b)))
    for i in range(3):
        for j in range(3):
            v = idx[3 * i:3 * i + 3, 3 * j:3 * j + 3].reshape(-1)
            for a in v:
                for b in v:
                    if a != b:
                        es.add((int(a), int(b)))
    return np.array(sorted(es), dtype=np.int32)


_EDGES = _build_edges_np()
_NE = _EDGES.shape[0]

_GSRC = np.zeros((EP, NP), np.float32)
_GDST = np.zeros((EP, NP), np.float32)
_GSRC[np.arange(_NE), _EDGES[:, 0]] = 1.0
_GDST[np.arange(_NE), _EDGES[:, 1]] = 1.0
_SCAT = _GDST.T.copy()

_ROWS = np.repeat(np.arange(9), 9)
_COLS = np.tile(np.arange(9), 9)


def _rrn_kernel(oh_ref, gsrc_ref, gdst_ref, scat_ref,
                vt_ref, pc_ref, wx2_ref, bx2_ref, wx3_ref, bx3_ref,
                wx4_ref, bx4_ref,
                w1a_ref, w1b_ref, b1_ref, w2_ref, b2_ref, w3_ref, b3_ref,
                w4_ref, b4_ref,
                wl1a_ref, wl1b_ref, bl1_ref, wl2_ref, bl2_ref, wl3_ref,
                bl3_ref, wl4_ref, bl4_ref,
                wih0_ref, wih1_ref, wih2_ref, wih3_ref,
                whh0_ref, whh1_ref, whh2_ref, whh3_ref,
                bg0_ref, bg1_ref, bg2_ref, bg3_ref,
                ro_ref, rb_ref, out_ref):
    oh = oh_ref[0]

    x = oh @ vt_ref[...] + pc_ref[...]
    x = jnp.maximum(x, 0.0)
    x = jnp.maximum(x @ wx2_ref[...] + bx2_ref[...], 0.0)
    x = jnp.maximum(x @ wx3_ref[...] + bx3_ref[...], 0.0)
    x = x @ wx4_ref[...] + bx4_ref[...]

    gsrc = gsrc_ref[...]
    gdst = gdst_ref[...]
    scat = scat_ref[...]

    h = x
    c = jnp.zeros_like(x)
    wih = (wih0_ref[...], wih1_ref[...], wih2_ref[...], wih3_ref[...])
    whh = (whh0_ref[...], whh1_ref[...], whh2_ref[...], whh3_ref[...])
    bg = (bg0_ref[...], bg1_ref[...], bg2_ref[...], bg3_ref[...])

    for t in range(NUM_STEPS):
        a = h @ w1a_ref[...]
        bv = h @ w1b_ref[...]
        e = gsrc @ a + gdst @ bv + b1_ref[...]
        e = jnp.maximum(e, 0.0)
        e = jnp.maximum(e @ w2_ref[...] + b2_ref[...], 0.0)
        e = jnp.maximum(e @ w3_ref[...] + b3_ref[...], 0.0)
        e = e @ w4_ref[...] + b4_ref[...]
        fm = scat @ e

        l = fm @ wl1a_ref[...] + x @ wl1b_ref[...] + bl1_ref[...]
        l = jnp.maximum(l, 0.0)
        l = jnp.maximum(l @ wl2_ref[...] + bl2_ref[...], 0.0)
        l = jnp.maximum(l @ wl3_ref[...] + bl3_ref[...], 0.0)
        l = l @ wl4_ref[...] + bl4_ref[...]

        def gate(k):
            g = l @ wih[k] + bg[k]
            if t > 0:
                g = g + h @ whh[k]
            return g

        i_g = jax.nn.sigmoid(gate(0))
        f_g = jax.nn.sigmoid(gate(1))
        g_g = jnp.tanh(gate(2))
        o_g = jax.nn.sigmoid(gate(3))
        c = f_g * c + i_g * g_g
        h = o_g * jnp.tanh(c)

        out_ref[t, 0] = h @ ro_ref[...] + rb_ref[...]


def kernel(inp, params):
    f32 = jnp.float32

    e1w, e1b = params['e1']
    e2w, e2b = params['e2']
    e3w, e3b = params['e3']
    wx1, bx1 = params['ex'][0]
    vt = e1w @ wx1[:16]
    vt = jnp.pad(vt, ((0, 6), (0, 0)))
    emb2 = e2w[_ROWS] + e2b
    emb3 = e3w[_COLS] + e3b
    pc = emb2 @ wx1[16:32] + emb3 @ wx1[32:48] + bx1 + e1b @ wx1[:16]
    pc = jnp.pad(pc, ((0, NP - 81), (0, 0)))

    (wx2, bx2), (wx3, bx3), (wx4, bx4) = params['ex'][1:]

    (w1, b1), (w2, b2), (w3, b3), (w4, b4) = params['mm']
    w1a, w1b = w1[:H], w1[H:]
    (wl1, bl1), (wl2, bl2), (wl3, bl3), (wl4, bl4) = params['ml']
    wl1a, wl1b = wl1[:H], wl1[H:]

    wih = params['wih']
    whh = params['whh']
    bgsum = params['bih'] + params['bhh']
    wih_k = [wih[:, k * H:(k + 1) * H] for k in range(4)]
    whh_k = [whh[:, k * H:(k + 1) * H] for k in range(4)]
    bg_k = [bgsum[k * H:(k + 1) * H].reshape(1, H) for k in range(4)]

    row = lambda v: v.reshape(1, -1)

    ro = jnp.pad(params['ro'][0], ((0, 0), (0, 6)))
    rb = jnp.pad(params['ro'][1], ((0, 6),)).reshape(1, 16)

    oh = jax.nn.one_hot(inp, 10, dtype=f32)
    oh = jnp.pad(oh, ((0, 0), (0, NP - 81), (0, 6)))

    gsrc = jnp.asarray(_GSRC)
    gdst = jnp.asarray(_GDST)
    scat = jnp.asarray(_SCAT)

    def cspec(shape):
        nd = len(shape)
        return pl.BlockSpec(shape, lambda i, _n=nd: (0,) * _n)

    operands = [
        oh, gsrc, gdst, scat,
        vt, pc, wx2, row(bx2), wx3, row(bx3), wx4, row(bx4),
        w1a, w1b, row(b1), w2, row(b2), w3, row(b3), w4, row(b4),
        wl1a, wl1b, row(bl1), wl2, row(bl2), wl3, row(bl3), wl4, row(bl4),
        *wih_k, *whh_k, *bg_k,
        ro, rb,
    ]
    in_specs = [pl.BlockSpec((1, NP, 16), lambda i: (i, 0, 0))]
    in_specs += [cspec(op.shape) for op in operands[1:]]

    out = pl.pallas_call(
        _rrn_kernel,
        grid=(B,),
        in_specs=in_specs,
        out_specs=pl.BlockSpec((NUM_STEPS, 1, NP, 16), lambda i: (0, i, 0, 0)),
        out_shape=jax.ShapeDtypeStruct((NUM_STEPS, B, NP, 16), f32),
        compiler_params=pltpu.CompilerParams(
            dimension_semantics=("parallel",),
        ),
    )(*operands)

    return out[:, :, :81, :10].reshape(NUM_STEPS, B * 81, 10)


# speedup vs baseline: 6.3043x; 2.2953x over previous
import numpy as np
import jax
import jax.numpy as jnp
from jax.experimental import pallas as pl
from jax.experimental.pallas import tpu as pltpu

B = 256
H = 96
NP = 88
EP = 1624
TB = 8
NUM_STEPS = 4


def _build_edges_np():
    idx = np.arange(81).reshape(9, 9)
    es = set()
    for i in range(9):
        for a in idx[i, :]:
            for b in idx[i, :]:
                if a != b:
                    es.add((int(a), int(b)))
        for a in idx[:, i]:
            for b in idx[:, i]:
                if a != b:
                    es.add((int(a), int(b)))
    for i in range(3):
        for j in range(3):
            v = idx[3 * i:3 * i + 3, 3 * j:3 * j + 3].reshape(-1)
            for a in v:
                for b in v:
                    if a != b:
                        es.add((int(a), int(b)))
    return np.array(sorted(es), dtype=np.int32)


_EDGES = _build_edges_np()
_NE = _EDGES.shape[0]

_GSRC = np.zeros((EP, NP), np.float32)
_GDST = np.zeros((EP, NP), np.float32)
_GSRC[np.arange(_NE), _EDGES[:, 0]] = 1.0
_GDST[np.arange(_NE), _EDGES[:, 1]] = 1.0
_SCAT = _GDST.T.copy()

_ROWS = np.repeat(np.arange(9), 9)
_COLS = np.tile(np.arange(9), 9)


def _rrn_kernel(oh_ref, gsrc_ref, gdst_ref, scat_ref,
                vt_ref, pc_ref, wx2_ref, bx2_ref, wx3_ref, bx3_ref,
                wx4_ref, bx4_ref,
                w1a_ref, w1b_ref, b1_ref, w2_ref, b2_ref, w3_ref, b3_ref,
                w4_ref, b4_ref,
                wl1a_ref, wl1b_ref, bl1_ref, wl2_ref, bl2_ref, wl3_ref,
                bl3_ref, wl4_ref, bl4_ref,
                wih0_ref, wih1_ref, wih2_ref, wih3_ref,
                whh0_ref, whh1_ref, whh2_ref, whh3_ref,
                bg0_ref, bg1_ref, bg2_ref, bg3_ref,
                ro_ref, rb_ref, out_ref):
    oh = oh_ref[...].reshape(TB * NP, 16)

    x = oh @ vt_ref[...] + jnp.tile(pc_ref[...], (TB, 1))
    x = jnp.maximum(x, 0.0)
    x = jnp.maximum(x @ wx2_ref[...] + bx2_ref[...], 0.0)
    x = jnp.maximum(x @ wx3_ref[...] + bx3_ref[...], 0.0)
    x = x @ wx4_ref[...] + bx4_ref[...]

    gsrc = gsrc_ref[...]
    gdst = gdst_ref[...]
    scat = scat_ref[...]

    h = x
    c = jnp.zeros_like(x)
    wih = (wih0_ref[...], wih1_ref[...], wih2_ref[...], wih3_ref[...])
    whh = (whh0_ref[...], whh1_ref[...], whh2_ref[...], whh3_ref[...])
    bg = (bg0_ref[...], bg1_ref[...], bg2_ref[...], bg3_ref[...])

    for t in range(NUM_STEPS):
        a = h @ w1a_ref[...]
        bv = h @ w1b_ref[...]
        pre = [gsrc @ a[tb * NP:(tb + 1) * NP]
               + gdst @ bv[tb * NP:(tb + 1) * NP]
               for tb in range(TB)]
        e = jnp.concatenate(pre, axis=0) + b1_ref[...]
        e = jnp.maximum(e, 0.0)
        e = jnp.maximum(e @ w2_ref[...] + b2_ref[...], 0.0)
        e = jnp.maximum(e @ w3_ref[...] + b3_ref[...], 0.0)
        e = e @ w4_ref[...] + b4_ref[...]
        fm = jnp.concatenate(
            [scat @ e[tb * EP:(tb + 1) * EP] for tb in range(TB)],
            axis=0)

        l = fm @ wl1a_ref[...] + x @ wl1b_ref[...] + bl1_ref[...]
        l = jnp.maximum(l, 0.0)
        l = jnp.maximum(l @ wl2_ref[...] + bl2_ref[...], 0.0)
        l = jnp.maximum(l @ wl3_ref[...] + bl3_ref[...], 0.0)
        l = l @ wl4_ref[...] + bl4_ref[...]

        def gate(k):
            g = l @ wih[k] + bg[k]
            if t > 0:
                g = g + h @ whh[k]
            return g

        i_g = jax.nn.sigmoid(gate(0))
        f_g = jax.nn.sigmoid(gate(1))
        g_g = jnp.tanh(gate(2))
        o_g = jax.nn.sigmoid(gate(3))
        c = f_g * c + i_g * g_g
        h = o_g * jnp.tanh(c)

        out_ref[t] = (h @ ro_ref[...] + rb_ref[...]).reshape(TB, NP, 16)


def kernel(inp, params):
    f32 = jnp.float32

    e1w, e1b = params['e1']
    e2w, e2b = params['e2']
    e3w, e3b = params['e3']
    wx1, bx1 = params['ex'][0]
    vt = e1w @ wx1[:16]
    vt = jnp.pad(vt, ((0, 6), (0, 0)))
    emb2 = e2w[_ROWS] + e2b
    emb3 = e3w[_COLS] + e3b
    pc = emb2 @ wx1[16:32] + emb3 @ wx1[32:48] + bx1 + e1b @ wx1[:16]
    pc = jnp.pad(pc, ((0, NP - 81), (0, 0)))

    (wx2, bx2), (wx3, bx3), (wx4, bx4) = params['ex'][1:]

    (w1, b1), (w2, b2), (w3, b3), (w4, b4) = params['mm']
    w1a, w1b = w1[:H], w1[H:]
    (wl1, bl1), (wl2, bl2), (wl3, bl3), (wl4, bl4) = params['ml']
    wl1a, wl1b = wl1[:H], wl1[H:]

    wih = params['wih']
    whh = params['whh']
    bgsum = params['bih'] + params['bhh']
    wih_k = [wih[:, k * H:(k + 1) * H] for k in range(4)]
    whh_k = [whh[:, k * H:(k + 1) * H] for k in range(4)]
    bg_k = [bgsum[k * H:(k + 1) * H].reshape(1, H) for k in range(4)]

    row = lambda v: v.reshape(1, -1)

    ro = jnp.pad(params['ro'][0], ((0, 0), (0, 6)))
    rb = jnp.pad(params['ro'][1], ((0, 6),)).reshape(1, 16)

    oh = jax.nn.one_hot(inp, 10, dtype=f32)
    oh = jnp.pad(oh, ((0, 0), (0, NP - 81), (0, 6)))

    gsrc = jnp.asarray(_GSRC)
    gdst = jnp.asarray(_GDST)
    scat = jnp.asarray(_SCAT)

    def cspec(shape):
        nd = len(shape)
        return pl.BlockSpec(shape, lambda i, _n=nd: (0,) * _n)

    operands = [
        oh, gsrc, gdst, scat,
        vt, pc, wx2, row(bx2), wx3, row(bx3), wx4, row(bx4),
        w1a, w1b, row(b1), w2, row(b2), w3, row(b3), w4, row(b4),
        wl1a, wl1b, row(bl1), wl2, row(bl2), wl3, row(bl3), wl4, row(bl4),
        *wih_k, *whh_k, *bg_k,
        ro, rb,
    ]
    in_specs = [pl.BlockSpec((TB, NP, 16), lambda i: (i, 0, 0))]
    in_specs += [cspec(op.shape) for op in operands[1:]]

    out = pl.pallas_call(
        _rrn_kernel,
        grid=(B // TB,),
        in_specs=in_specs,
        out_specs=pl.BlockSpec((NUM_STEPS, TB, NP, 16),
                               lambda i: (0, i, 0, 0)),
        out_shape=jax.ShapeDtypeStruct((NUM_STEPS, B, NP, 16), f32),
        compiler_params=pltpu.CompilerParams(
            dimension_semantics=("parallel",),
        ),
    )(*operands)

    return out[:, :, :81, :10].reshape(NUM_STEPS, B * 81, 10)


# speedup vs baseline: 8.7670x; 1.3906x over previous
import numpy as np
import jax
import jax.numpy as jnp
from jax.experimental import pallas as pl
from jax.experimental.pallas import tpu as pltpu

B = 256
H = 96
NP = 88
EP = 1624
TB = 8
NUM_STEPS = 4


def _build_edges_np():
    idx = np.arange(81).reshape(9, 9)
    es = set()
    for i in range(9):
        for a in idx[i, :]:
            for b in idx[i, :]:
                if a != b:
                    es.add((int(a), int(b)))
        for a in idx[:, i]:
            for b in idx[:, i]:
                if a != b:
                    es.add((int(a), int(b)))
    for i in range(3):
        for j in range(3):
            v = idx[3 * i:3 * i + 3, 3 * j:3 * j + 3].reshape(-1)
            for a in v:
                for b in v:
                    if a != b:
                        es.add((int(a), int(b)))
    return np.array(sorted(es), dtype=np.int32)


_EDGES = _build_edges_np()
_NE = _EDGES.shape[0]
_DEG = 20
EP3 = _DEG * NP

_GSRC3 = np.zeros((EP3, NP), np.float32)
for _d in range(81):
    _srcs = _EDGES[_EDGES[:, 1] == _d, 0]
    assert _srcs.shape[0] == _DEG
    for _k, _s in enumerate(_srcs):
        _GSRC3[_k * NP + _d, _s] = 1.0

_ROWS = np.repeat(np.arange(9), 9)
_COLS = np.tile(np.arange(9), 9)


def _rrn_kernel(oh_ref, gsrc_ref,
                vt_ref, pc_ref, wx2_ref, bx2_ref, wx3_ref, bx3_ref,
                wx4_ref, bx4_ref,
                w1a_ref, w1b_ref, b1_ref, w2_ref, b2_ref, w3_ref, b3_ref,
                w4_ref, b4_ref,
                wl1a_ref, wl1b_ref, bl1_ref, wl2_ref, bl2_ref, wl3_ref,
                bl3_ref, wl4_ref, bl4_ref,
                wih0_ref, wih1_ref, wih2_ref, wih3_ref,
                whh0_ref, whh1_ref, whh2_ref, whh3_ref,
                bg0_ref, bg1_ref, bg2_ref, bg3_ref,
                ro_ref, rb_ref, out_ref):
    oh = oh_ref[...].reshape(TB * NP, 16)

    x = oh @ vt_ref[...] + jnp.tile(pc_ref[...], (TB, 1))
    x = jnp.maximum(x, 0.0)
    x = jnp.maximum(x @ wx2_ref[...] + bx2_ref[...], 0.0)
    x = jnp.maximum(x @ wx3_ref[...] + bx3_ref[...], 0.0)
    x = x @ wx4_ref[...] + bx4_ref[...]

    gsrc = gsrc_ref[...]

    h = x
    c = jnp.zeros_like(x)
    wih = (wih0_ref[...], wih1_ref[...], wih2_ref[...], wih3_ref[...])
    whh = (whh0_ref[...], whh1_ref[...], whh2_ref[...], whh3_ref[...])
    bg = (bg0_ref[...], bg1_ref[...], bg2_ref[...], bg3_ref[...])

    for t in range(NUM_STEPS):
        a = h @ w1a_ref[...]
        bv = h @ w1b_ref[...]
        pre = [gsrc @ a[tb * NP:(tb + 1) * NP] for tb in range(TB)]
        bvx = jnp.broadcast_to(
            bv.reshape(TB, 1, NP, H), (TB, _DEG, NP, H)
        ).reshape(TB * EP3, H)
        e = jnp.concatenate(pre, axis=0) + bvx + b1_ref[...]
        e = jnp.maximum(e, 0.0)
        e = jnp.maximum(e @ w2_ref[...] + b2_ref[...], 0.0)
        e = jnp.maximum(e @ w3_ref[...] + b3_ref[...], 0.0)
        e = e @ w4_ref[...] + b4_ref[...]
        fm = jnp.sum(e.reshape(TB, _DEG, NP, H), axis=1).reshape(TB * NP, H)

        l = fm @ wl1a_ref[...] + x @ wl1b_ref[...] + bl1_ref[...]
        l = jnp.maximum(l, 0.0)
        l = jnp.maximum(l @ wl2_ref[...] + bl2_ref[...], 0.0)
        l = jnp.maximum(l @ wl3_ref[...] + bl3_ref[...], 0.0)
        l = l @ wl4_ref[...] + bl4_ref[...]

        def gate(k):
            g = l @ wih[k] + bg[k]
            if t > 0:
                g = g + h @ whh[k]
            return g

        i_g = jax.nn.sigmoid(gate(0))
        f_g = jax.nn.sigmoid(gate(1))
        g_g = jnp.tanh(gate(2))
        o_g = jax.nn.sigmoid(gate(3))
        c = f_g * c + i_g * g_g
        h = o_g * jnp.tanh(c)

        out_ref[t] = (h @ ro_ref[...] + rb_ref[...]).reshape(TB, NP, 16)


def kernel(inp, params):
    f32 = jnp.float32

    e1w, e1b = params['e1']
    e2w, e2b = params['e2']
    e3w, e3b = params['e3']
    wx1, bx1 = params['ex'][0]
    vt = e1w @ wx1[:16]
    vt = jnp.pad(vt, ((0, 6), (0, 0)))
    emb2 = e2w[_ROWS] + e2b
    emb3 = e3w[_COLS] + e3b
    pc = emb2 @ wx1[16:32] + emb3 @ wx1[32:48] + bx1 + e1b @ wx1[:16]
    pc = jnp.pad(pc, ((0, NP - 81), (0, 0)))

    (wx2, bx2), (wx3, bx3), (wx4, bx4) = params['ex'][1:]

    (w1, b1), (w2, b2), (w3, b3), (w4, b4) = params['mm']
    w1a, w1b = w1[:H], w1[H:]
    (wl1, bl1), (wl2, bl2), (wl3, bl3), (wl4, bl4) = params['ml']
    wl1a, wl1b = wl1[:H], wl1[H:]

    wih = params['wih']
    whh = params['whh']
    bgsum = params['bih'] + params['bhh']
    wih_k = [wih[:, k * H:(k + 1) * H] for k in range(4)]
    whh_k = [whh[:, k * H:(k + 1) * H] for k in range(4)]
    bg_k = [bgsum[k * H:(k + 1) * H].reshape(1, H) for k in range(4)]

    row = lambda v: v.reshape(1, -1)

    ro = jnp.pad(params['ro'][0], ((0, 0), (0, 6)))
    rb = jnp.pad(params['ro'][1], ((0, 6),)).reshape(1, 16)

    oh = jax.nn.one_hot(inp, 10, dtype=f32)
    oh = jnp.pad(oh, ((0, 0), (0, NP - 81), (0, 6)))

    gsrc = jnp.asarray(_GSRC3)

    def cspec(shape):
        nd = len(shape)
        return pl.BlockSpec(shape, lambda i, _n=nd: (0,) * _n)

    operands = [
        oh, gsrc,
        vt, pc, wx2, row(bx2), wx3, row(bx3), wx4, row(bx4),
        w1a, w1b, row(b1), w2, row(b2), w3, row(b3), w4, row(b4),
        wl1a, wl1b, row(bl1), wl2, row(bl2), wl3, row(bl3), wl4, row(bl4),
        *wih_k, *whh_k, *bg_k,
        ro, rb,
    ]
    in_specs = [pl.BlockSpec((TB, NP, 16), lambda i: (i, 0, 0))]
    in_specs += [cspec(op.shape) for op in operands[1:]]

    out = pl.pallas_call(
        _rrn_kernel,
        grid=(B // TB,),
        in_specs=in_specs,
        out_specs=pl.BlockSpec((NUM_STEPS, TB, NP, 16),
                               lambda i: (0, i, 0, 0)),
        out_shape=jax.ShapeDtypeStruct((NUM_STEPS, B, NP, 16), f32),
        compiler_params=pltpu.CompilerParams(
            dimension_semantics=("parallel",),
        ),
    )(*operands)

    return out[:, :, :81, :10].reshape(NUM_STEPS, B * 81, 10)
